# baseline (speedup 1.0000x reference)
import functools

import jax
import jax.numpy as jnp
from jax import lax
from jax.experimental import pallas as pl
from jax.experimental.pallas import tpu as pltpu
from jax.experimental.pallas import tpu_sc as plsc

B = 16
CIN = 768
T = 576
CD = 64
K = 8192
KS = 512

NC = 2
NS = 16
NW = NC * NS
ROWS = B * T
BPW = ROWS // NW
NCH = 3
CH = BPW // NCH
CDP = 128


def _encode_body(z_ref, w_ref, b_ref, cbt_ref, cb_ref, ze_ref, idx_ref,
                 cbp_ref):
    @pl.when(pl.program_id(0) == 0)
    def _pad_table():
        cbp_ref[:, :CD] = cb_ref[...]
        cbp_ref[:, CD:] = jnp.zeros((K, CDP - CD), jnp.float32)

    z = z_ref[0]
    w = w_ref[...]
    ze_t = lax.dot_general(z, w, (((0,), (1,)), ((), ())),
                           preferred_element_type=jnp.float32)
    ze_t = ze_t + b_ref[...]
    ze_ref[0] = ze_t.T
    n = jnp.sqrt(jnp.sum(ze_t * ze_t, axis=1, keepdims=True))
    enc = ze_t / jnp.maximum(n, 1e-12)
    enc_aug = jnp.concatenate(
        [enc, jnp.ones((T, 1), jnp.float32)], axis=1)
    rmax = None
    rgs = None
    for j in range(K // KS):
        s = lax.dot_general(enc_aug, cbt_ref[:, pl.ds(j * KS, KS)],
                            (((1,), (0,)), ((), ())),
                            preferred_element_type=jnp.float32)
        s0, s1 = s[:, 0:128], s[:, 128:256]
        s2, s3 = s[:, 256:384], s[:, 384:512]
        m01 = jnp.maximum(s0, s1)
        g01 = jnp.where(s1 > s0, jnp.int32(4 * j + 1), jnp.int32(4 * j))
        m23 = jnp.maximum(s2, s3)
        g23 = jnp.where(s3 > s2, jnp.int32(4 * j + 3), jnp.int32(4 * j + 2))
        m = jnp.maximum(m01, m23)
        gs = jnp.where(m23 > m01, g23, g01)
        if rmax is None:
            rmax, rgs = m, gs
        else:
            upd = m > rmax
            rmax = jnp.maximum(m, rmax)
            rgs = jnp.where(upd, gs, rgs)
    rmax_t = rmax.T
    kvec_t = rgs.T * 128 + lax.broadcasted_iota(jnp.int32, (128, T), 0)
    fmax_t = jnp.max(rmax_t, axis=0, keepdims=True)
    fidx_t = jnp.min(jnp.where(rmax_t == fmax_t, kvec_t, jnp.int32(K)),
                     axis=0, keepdims=True)
    idx_ref[0] = fidx_t


def _decode_body(zq_ref, ze_ref, w_ref, b_ref, out_ref, loss_ref):
    zq = zq_ref[0, :, :CD]
    ze = ze_ref[0]
    w = w_ref[...]
    o = lax.dot_general(w, zq, (((1,), (1,)), ((), ())),
                        preferred_element_type=jnp.float32)
    out_ref[0] = o + b_ref[...]
    d = ze - zq.T
    loss_ref[0] = jnp.full((1, 128), jnp.sum(d * d), jnp.float32)


@functools.cache
def _make_gather_rows():
    mesh = plsc.VectorSubcoreMesh(core_axis_name="c", subcore_axis_name="s")

    @functools.partial(
        pl.kernel,
        mesh=mesh,
        out_type=jax.ShapeDtypeStruct((ROWS, CDP), jnp.float32),
        scratch_types=[
            pltpu.VMEM((NCH, CH), jnp.int32),
            pltpu.VMEM((BPW, CDP), jnp.float32),
            pltpu.SemaphoreType.DMA,
        ],
    )
    def _gather_rows(idx_hbm, table_hbm, out_hbm, idx_v, rows_v, sem):
        wid = lax.axis_index("s") * NC + lax.axis_index("c")
        pltpu.sync_copy(idx_hbm.at[wid], idx_v)
        copies = []
        for j in range(NCH):
            copies.append(pltpu.async_copy(
                table_hbm.at[idx_v.at[j]],
                rows_v.at[pl.ds(j * CH, CH)], sem))
        for c in copies:
            c.wait()
        pltpu.sync_copy(rows_v, out_hbm.at[pl.ds(wid * BPW, BPW)])

    return _gather_rows


def _wn(v, g):
    n = jnp.sqrt(jnp.sum(v * v, axis=1, keepdims=True))
    return g[:, None] * v / jnp.maximum(n, 1e-12)


def kernel(z, in_v, in_g, in_b, out_v, out_g, out_b, codebook):
    w_in = _wn(in_v, in_g)
    w_out = _wn(out_v, out_g)
    cbn = codebook / jnp.maximum(
        jnp.sqrt(jnp.sum(codebook * codebook, axis=1, keepdims=True)), 1e-12)
    cbt_aug = jnp.concatenate(
        [cbn.T, -0.5 * jnp.sum(cbn * cbn, axis=1)[None, :]], axis=0)

    ze, idx3, cb_pad = pl.pallas_call(
        _encode_body,
        grid=(B,),
        in_specs=[
            pl.BlockSpec((1, CIN, T), lambda b: (b, 0, 0)),
            pl.BlockSpec((CD, CIN), lambda b: (0, 0)),
            pl.BlockSpec((1, CD), lambda b: (0, 0)),
            pl.BlockSpec((CD + 1, K), lambda b: (0, 0)),
            pl.BlockSpec((K, CD), lambda b: (0, 0)),
        ],
        out_specs=[
            pl.BlockSpec((1, CD, T), lambda b: (b, 0, 0)),
            pl.BlockSpec((1, 1, T), lambda b: (b, 0, 0)),
            pl.BlockSpec((K, CDP), lambda b: (0, 0)),
        ],
        out_shape=[
            jax.ShapeDtypeStruct((B, CD, T), jnp.float32),
            jax.ShapeDtypeStruct((B, 1, T), jnp.int32),
            jax.ShapeDtypeStruct((K, CDP), jnp.float32),
        ],
    )(z, w_in, in_b.reshape(1, CD), cbt_aug, codebook)

    indices = idx3.reshape(B, T)
    zq_rows = _make_gather_rows()(indices.reshape(NW, NCH, CH), cb_pad)

    out, loss3 = pl.pallas_call(
        _decode_body,
        grid=(B,),
        in_specs=[
            pl.BlockSpec((1, T, CDP), lambda b: (b, 0, 0)),
            pl.BlockSpec((1, CD, T), lambda b: (b, 0, 0)),
            pl.BlockSpec((CIN, CD), lambda b: (0, 0)),
            pl.BlockSpec((CIN, 1), lambda b: (0, 0)),
        ],
        out_specs=[
            pl.BlockSpec((1, CIN, T), lambda b: (b, 0, 0)),
            pl.BlockSpec((1, 1, 128), lambda b: (b, 0, 0)),
        ],
        out_shape=[
            jax.ShapeDtypeStruct((B, CIN, T), jnp.float32),
            jax.ShapeDtypeStruct((B, 1, 128), jnp.float32),
        ],
    )(zq_rows.reshape(B, T, CDP), ze, w_out, out_b.reshape(CIN, 1))

    loss = loss3[:, 0, 0] * (1.0 / (CD * T))
    return out, loss, loss, indices, ze


# speedup vs baseline: 1.0088x; 1.0088x over previous
import functools

import jax
import jax.numpy as jnp
from jax import lax
from jax.experimental import pallas as pl
from jax.experimental.pallas import tpu as pltpu
from jax.experimental.pallas import tpu_sc as plsc

B = 16
CIN = 768
T = 576
CD = 64
K = 8192
KS = 256

NC = 2
NS = 16
NW = NC * NS
ROWS = B * T
BPW = ROWS // NW
NCH = 3
CH = BPW // NCH
CDP = 128


def _encode_body(z_ref, w_ref, b_ref, cbt_ref, ze_ref, idx_ref):
    z = z_ref[0]
    w = w_ref[...]
    ze_t = lax.dot_general(z, w, (((0,), (1,)), ((), ())),
                           preferred_element_type=jnp.float32)
    ze_t = ze_t + b_ref[...]
    ze_ref[0] = ze_t.T
    n = jnp.sqrt(jnp.sum(ze_t * ze_t, axis=1, keepdims=True))
    enc = ze_t / jnp.maximum(n, 1e-12)
    enc_aug = jnp.concatenate(
        [enc, jnp.ones((T, 1), jnp.float32)], axis=1)
    rmax = None
    rgs = None
    for j in range(K // KS):
        s = lax.dot_general(enc_aug, cbt_ref[:, pl.ds(j * KS, KS)],
                            (((1,), (0,)), ((), ())),
                            preferred_element_type=jnp.float32)
        NG = KS // 128
        vals = [s[:, g * 128:(g + 1) * 128] for g in range(NG)]
        ids = [jnp.int32(NG * j + g) for g in range(NG)]
        while len(vals) > 1:
            nv, ni = [], []
            for a in range(0, len(vals), 2):
                lo, hi = vals[a], vals[a + 1]
                w_hi = hi > lo
                nv.append(jnp.maximum(lo, hi))
                ni.append(jnp.where(w_hi, ids[a + 1], ids[a]))
            vals, ids = nv, ni
        m, gs = vals[0], ids[0]
        if rmax is None:
            rmax, rgs = m, gs
        else:
            upd = m > rmax
            rmax = jnp.maximum(m, rmax)
            rgs = jnp.where(upd, gs, rgs)
    rmax_t = rmax.T
    kvec_t = rgs.T * 128 + lax.broadcasted_iota(jnp.int32, (128, T), 0)
    fmax_t = jnp.max(rmax_t, axis=0, keepdims=True)
    fidx_t = jnp.min(jnp.where(rmax_t == fmax_t, kvec_t, jnp.int32(K)),
                     axis=0, keepdims=True)
    idx_ref[0] = fidx_t


def _decode_body(zq_ref, ze_ref, w_ref, b_ref, out_ref, loss_ref):
    zq = zq_ref[0, :, :CD]
    ze = ze_ref[0]
    w = w_ref[...]
    o = lax.dot_general(w, zq, (((1,), (1,)), ((), ())),
                        preferred_element_type=jnp.float32)
    out_ref[0] = o + b_ref[...]
    d = ze - zq.T
    loss_ref[0] = jnp.full((1, 128), jnp.sum(d * d), jnp.float32)


@functools.cache
def _make_gather_rows():
    mesh = plsc.VectorSubcoreMesh(core_axis_name="c", subcore_axis_name="s")

    @functools.partial(
        pl.kernel,
        mesh=mesh,
        out_type=jax.ShapeDtypeStruct((ROWS, CDP), jnp.float32),
        scratch_types=[
            pltpu.VMEM((NCH, CH), jnp.int32),
            pltpu.VMEM((BPW, CDP), jnp.float32),
            pltpu.SemaphoreType.DMA,
        ],
    )
    def _gather_rows(idx_hbm, table_hbm, out_hbm, idx_v, rows_v, sem):
        wid = lax.axis_index("s") * NC + lax.axis_index("c")
        pltpu.sync_copy(idx_hbm.at[wid], idx_v)
        copies = []
        for j in range(NCH):
            copies.append(pltpu.async_copy(
                table_hbm.at[idx_v.at[j]],
                rows_v.at[pl.ds(j * CH, CH)], sem))
        for c in copies:
            c.wait()
        pltpu.sync_copy(rows_v, out_hbm.at[pl.ds(wid * BPW, BPW)])

    return _gather_rows


def _wn(v, g):
    n = jnp.sqrt(jnp.sum(v * v, axis=1, keepdims=True))
    return g[:, None] * v / jnp.maximum(n, 1e-12)


def kernel(z, in_v, in_g, in_b, out_v, out_g, out_b, codebook):
    w_in = _wn(in_v, in_g)
    w_out = _wn(out_v, out_g)
    cbn = codebook / jnp.maximum(
        jnp.sqrt(jnp.sum(codebook * codebook, axis=1, keepdims=True)), 1e-12)
    cbt_aug = jnp.concatenate(
        [cbn.T, -0.5 * jnp.sum(cbn * cbn, axis=1)[None, :]], axis=0)

    ze, idx3 = pl.pallas_call(
        _encode_body,
        grid=(B,),
        in_specs=[
            pl.BlockSpec((1, CIN, T), lambda b: (b, 0, 0)),
            pl.BlockSpec((CD, CIN), lambda b: (0, 0)),
            pl.BlockSpec((1, CD), lambda b: (0, 0)),
            pl.BlockSpec((CD + 1, K), lambda b: (0, 0)),
        ],
        out_specs=[
            pl.BlockSpec((1, CD, T), lambda b: (b, 0, 0)),
            pl.BlockSpec((1, 1, T), lambda b: (b, 0, 0)),
        ],
        out_shape=[
            jax.ShapeDtypeStruct((B, CD, T), jnp.float32),
            jax.ShapeDtypeStruct((B, 1, T), jnp.int32),
        ],
    )(z, w_in, in_b.reshape(1, CD), cbt_aug)

    indices = idx3.reshape(B, T)
    cb_pad = jnp.pad(codebook, ((0, 0), (0, CDP - CD)))
    zq_rows = _make_gather_rows()(indices.reshape(NW, NCH, CH), cb_pad)

    out, loss3 = pl.pallas_call(
        _decode_body,
        grid=(B,),
        in_specs=[
            pl.BlockSpec((1, T, CDP), lambda b: (b, 0, 0)),
            pl.BlockSpec((1, CD, T), lambda b: (b, 0, 0)),
            pl.BlockSpec((CIN, CD), lambda b: (0, 0)),
            pl.BlockSpec((CIN, 1), lambda b: (0, 0)),
        ],
        out_specs=[
            pl.BlockSpec((1, CIN, T), lambda b: (b, 0, 0)),
            pl.BlockSpec((1, 1, 128), lambda b: (b, 0, 0)),
        ],
        out_shape=[
            jax.ShapeDtypeStruct((B, CIN, T), jnp.float32),
            jax.ShapeDtypeStruct((B, 1, 128), jnp.float32),
        ],
    )(zq_rows.reshape(B, T, CDP), ze, w_out, out_b.reshape(CIN, 1))

    loss = loss3[:, 0, 0] * (1.0 / (CD * T))
    return out, loss, loss, indices, ze
